# Initial kernel scaffold; baseline (speedup 1.0000x reference)
import jax
import jax.numpy as jnp
from jax.experimental import pallas as pl


def kernel(x, Wg, We, be):
    raise NotImplementedError("write your pallas kernel here")


# speedup vs baseline: 4.2760x; 4.2760x over previous
import functools

import jax
import jax.numpy as jnp
from jax.experimental import pallas as pl


def _moe_body(x_ref, wg_ref, we_ref, bet_ref, out_ref, gate_ref, *, nb, e, out_len):
    b = pl.program_id(0)
    xb = x_ref[0]
    nv = xb.shape[1]

    logits = jnp.dot(wg_ref[...], xb, preferred_element_type=jnp.float32)
    m = jnp.max(logits, axis=0, keepdims=True)
    ex = jnp.exp(logits - m)
    g = ex / jnp.sum(ex, axis=0, keepdims=True)

    iota = jax.lax.broadcasted_iota(jnp.int32, (e, nv), 0)
    m1 = jnp.max(g, axis=0, keepdims=True)
    idx1 = jnp.min(jnp.where(g == m1, iota, e), axis=0, keepdims=True)
    mask1 = iota == idx1
    neg = jnp.float32(-jnp.inf)
    g2 = jnp.where(mask1, neg, g)
    m2 = jnp.max(g2, axis=0, keepdims=True)
    idx2 = jnp.min(jnp.where(g2 == m2, iota, e), axis=0, keepdims=True)
    mask2 = iota == idx2
    wd = jnp.where(mask1 | mask2, g, 0.0)

    y = jnp.dot(we_ref[...], xb, preferred_element_type=jnp.float32)
    acc = jnp.dot(bet_ref[...], wd, preferred_element_type=jnp.float32)
    for i in range(e):
        acc = acc + wd[i : i + 1, :] * y[i * out_len : (i + 1) * out_len, :]
    out_ref[0] = acc

    @pl.when(b == 0)
    def _init():
        gate_ref[...] = jnp.zeros_like(gate_ref)

    gate_ref[...] += g

    @pl.when(b == nb - 1)
    def _fin():
        gate_ref[...] = gate_ref[...] * (1.0 / nb)


@jax.jit
def kernel(x, Wg, We, be):
    B, IN_LEN, NVARS = x.shape
    E, OUT_LEN, _ = We.shape

    we_flat = We.reshape(E * OUT_LEN, IN_LEN)
    be_t = be.T

    body = functools.partial(_moe_body, nb=B, e=E, out_len=OUT_LEN)
    out, gate_sum = pl.pallas_call(
        body,
        grid=(B,),
        in_specs=[
            pl.BlockSpec((1, IN_LEN, NVARS), lambda b: (b, 0, 0)),
            pl.BlockSpec((E, IN_LEN), lambda b: (0, 0)),
            pl.BlockSpec((E * OUT_LEN, IN_LEN), lambda b: (0, 0)),
            pl.BlockSpec((OUT_LEN, E), lambda b: (0, 0)),
        ],
        out_specs=[
            pl.BlockSpec((1, OUT_LEN, NVARS), lambda b: (b, 0, 0)),
            pl.BlockSpec((E, NVARS), lambda b: (0, 0)),
        ],
        out_shape=[
            jax.ShapeDtypeStruct((B, OUT_LEN, NVARS), x.dtype),
            jax.ShapeDtypeStruct((E, NVARS), jnp.float32),
        ],
    )(x, Wg, we_flat, be_t)

    gate_mean = gate_sum.T
    return (out, gate_mean)
